# baseline (speedup 1.0000x reference)
import functools

import jax
import jax.numpy as jnp
from jax import lax
from jax.experimental import pallas as pl
from jax.experimental.pallas import tpu as pltpu
from jax.experimental.pallas import tpu_sc as plsc

DIM = 128
NC = 2
NS = 16
NW = NC * NS

ROW_BLOCK = 4000
CHUNK = 128


def _transform_body(table_ref, w_ref, b_ref, out_ref):
    y = jnp.dot(table_ref[...], w_ref[...], preferred_element_type=jnp.float32)
    y = y + b_ref[...]
    out_ref[...] = jnp.where(y >= 0, y, 0.01 * y)


def _transform_table(table, W, b):
    n = table.shape[0]
    return pl.pallas_call(
        _transform_body,
        grid=(n // ROW_BLOCK,),
        in_specs=[
            pl.BlockSpec((ROW_BLOCK, DIM), lambda i: (i, 0)),
            pl.BlockSpec((DIM, DIM), lambda i: (0, 0)),
            pl.BlockSpec((1, DIM), lambda i: (0, 0)),
        ],
        out_specs=pl.BlockSpec((ROW_BLOCK, DIM), lambda i: (i, 0)),
        out_shape=jax.ShapeDtypeStruct((n, DIM), jnp.float32),
    )(table, W, b.reshape(1, DIM))


@functools.lru_cache(maxsize=None)
def _make_gather(n_idx):
    per_w = n_idx // NW
    n_chunks = per_w // CHUNK
    mesh = plsc.VectorSubcoreMesh(core_axis_name="c", subcore_axis_name="s")

    @functools.partial(
        pl.kernel,
        out_type=jax.ShapeDtypeStruct((n_idx, DIM), jnp.float32),
        mesh=mesh,
        scratch_types=[
            pltpu.VMEM((n_chunks, CHUNK), jnp.int32),
            pltpu.VMEM((CHUNK, DIM), jnp.float32),
            pltpu.SemaphoreType.DMA,
        ],
    )
    def gather(t_hbm, idx_hbm, out_hbm, idx_v, rows_v, sem):
        wid = lax.axis_index("s") * NC + lax.axis_index("c")
        base = wid * per_w
        pltpu.sync_copy(idx_hbm.at[wid], idx_v)

        def step(g, carry):
            pltpu.async_copy(t_hbm.at[idx_v.at[g]], rows_v, sem).wait()
            off = pl.multiple_of(base + g * CHUNK, CHUNK)
            pltpu.sync_copy(rows_v, out_hbm.at[pl.ds(off, CHUNK)])
            return carry

        lax.fori_loop(0, n_chunks, step, 0)

    return gather


def kernel(x, table, W, b):
    t = _transform_table(table, W, b)
    bsz, fields = x.shape
    fpad = 32
    xp = jnp.pad(x.astype(jnp.int32), ((0, 0), (0, fpad - fields)))
    n_idx = bsz * fpad
    idx = xp.reshape(NW, n_idx // (NW * CHUNK), CHUNK)
    out = _make_gather(n_idx)(t, idx)
    return out.reshape(bsz, fpad, DIM)[:, :fields, :]


# speedup vs baseline: 7.1588x; 7.1588x over previous
import functools

import jax
import jax.numpy as jnp
from jax import lax
from jax.experimental import pallas as pl
from jax.experimental.pallas import tpu as pltpu
from jax.experimental.pallas import tpu_sc as plsc

DIM = 128
NC = 2
NS = 16
NW = NC * NS

ROW_BLOCK = 4000
CHUNK = 128


def _transform_body(table_ref, w_ref, b_ref, out_ref):
    y = jnp.dot(table_ref[...], w_ref[...], preferred_element_type=jnp.float32)
    y = y + b_ref[...]
    out_ref[...] = jnp.where(y >= 0, y, 0.01 * y)


def _transform_table(table, W, b):
    n = table.shape[0]
    return pl.pallas_call(
        _transform_body,
        grid=(n // ROW_BLOCK,),
        in_specs=[
            pl.BlockSpec((ROW_BLOCK, DIM), lambda i: (i, 0)),
            pl.BlockSpec((DIM, DIM), lambda i: (0, 0)),
            pl.BlockSpec((1, DIM), lambda i: (0, 0)),
        ],
        out_specs=pl.BlockSpec((ROW_BLOCK, DIM), lambda i: (i, 0)),
        out_shape=jax.ShapeDtypeStruct((n, DIM), jnp.float32),
    )(table, W, b.reshape(1, DIM))


NBUF = 4


@functools.lru_cache(maxsize=None)
def _make_gather(n_idx):
    per_w = n_idx // NW
    n_chunks = per_w // CHUNK
    assert n_chunks % NBUF == 0
    mesh = plsc.VectorSubcoreMesh(core_axis_name="c", subcore_axis_name="s")

    @functools.partial(
        pl.kernel,
        out_type=jax.ShapeDtypeStruct((n_idx, DIM), jnp.float32),
        mesh=mesh,
        scratch_types=[
            pltpu.VMEM((n_chunks, CHUNK), jnp.int32),
            pltpu.VMEM((NBUF, CHUNK, DIM), jnp.float32),
        ]
        + [pltpu.SemaphoreType.DMA] * NBUF,
    )
    def gather(t_hbm, idx_hbm, out_hbm, idx_v, rows_v, *sems):
        wid = lax.axis_index("s") * NC + lax.axis_index("c")
        base = wid * per_w
        pltpu.sync_copy(idx_hbm.at[wid], idx_v)

        for b in range(NBUF):
            pltpu.async_copy(t_hbm.at[idx_v.at[b]], rows_v.at[b], sems[b])

        def step(i, carry):
            for b in range(NBUF):
                g = i * NBUF + b
                pltpu.make_async_copy(
                    t_hbm.at[idx_v.at[b]], rows_v.at[b], sems[b]
                ).wait()
                off = pl.multiple_of(base + g * CHUNK, CHUNK)
                pltpu.sync_copy(rows_v.at[b], out_hbm.at[pl.ds(off, CHUNK)])
                pltpu.async_copy(
                    t_hbm.at[idx_v.at[g + NBUF]], rows_v.at[b], sems[b]
                )
            return carry

        lax.fori_loop(0, n_chunks // NBUF - 1, step, 0)

        for b in range(NBUF):
            g = n_chunks - NBUF + b
            pltpu.make_async_copy(
                t_hbm.at[idx_v.at[b]], rows_v.at[b], sems[b]
            ).wait()
            off = pl.multiple_of(base + g * CHUNK, CHUNK)
            pltpu.sync_copy(rows_v.at[b], out_hbm.at[pl.ds(off, CHUNK)])

    return gather


def kernel(x, table, W, b):
    t = _transform_table(table, W, b)
    bsz, fields = x.shape
    n_idx = bsz * fields
    idx = x.reshape(NW, n_idx // (NW * CHUNK), CHUNK).astype(jnp.int32)
    out = _make_gather(n_idx)(t, idx)
    return out.reshape(bsz, fields, DIM)
